# baseline (speedup 1.0000x reference)
import functools

import jax
import jax.numpy as jnp
from jax import lax
from jax.experimental import pallas as pl
from jax.experimental.pallas import tpu as pltpu
from jax.experimental.pallas import tpu_sc as plsc

NUM_EMBEDDINGS = 1024
CODE_DIM = 256
NUM_CODEBOOKS = 4
COMMITMENT_COST = 0.25
EPS = 1e-10

def _vq_tc_body(total_flat_rows, z_ref, e_ref, counts_in_ref, commit_in_ref,
                idx_ref, counts_out_ref, commit_ref, ppl_ref, counts_scr,
                commit_scr):
    i = pl.program_id(0)
    e = e_ref[...]
    K, Dc = e.shape
    Rb = z_ref.shape[0]
    e2 = jnp.sum(e * e, axis=1, keepdims=True).reshape(1, -1)
    iota_f = lax.broadcasted_iota(jnp.int32, (1, K), 1).astype(jnp.float32)
    ones_r = jnp.ones((1, Rb), jnp.float32)
    idx_cols = []
    tile_counts = jnp.zeros((1, K), jnp.float32)
    tile_commit = jnp.zeros((), jnp.float32)
    for c in range(NUM_CODEBOOKS):
        zc = z_ref[:, c * Dc:(c + 1) * Dc]
        z2 = jnp.sum(zc * zc, axis=1, keepdims=True)
        s_neg = lax.dot_general(zc * -2.0, e, (((1,), (1,)), ((), ())),
                                preferred_element_type=jnp.float32)
        d = (z2 + s_neg) + e2
        md = jnp.min(d, axis=1, keepdims=True)
        idxf = jnp.min(jnp.where(d == md, iota_f, 2048.0), axis=1,
                       keepdims=True)
        idx_cols.append(idxf.astype(jnp.int32))
        onehot = (iota_f == idxf).astype(jnp.float32)
        tile_counts += lax.dot_general(ones_r, onehot,
                                       (((1,), (0,)), ((), ())),
                                       preferred_element_type=jnp.float32)
        tile_commit += jnp.sum(md)
    idx_ref[...] = jnp.concatenate(idx_cols, axis=1)

    @pl.when(i == 0)
    def _init():
        counts_scr[...] = tile_counts
        commit_scr[0] = tile_commit

    @pl.when(i > 0)
    def _acc():
        counts_scr[...] += tile_counts
        commit_scr[0] += tile_commit

    @pl.when(i == pl.num_programs(0) - 1)
    def _fin():
        counts_total = counts_scr[...] + counts_in_ref[...]
        commit_total = commit_scr[0] + commit_in_ref[0, 0]
        counts_out_ref[...] = counts_total
        commit_ref[...] = jnp.full((1, 1), commit_total, jnp.float32)
        p = counts_total / total_flat_rows
        ent = -jnp.sum(p * jnp.log(p + EPS), axis=1, keepdims=True)
        ppl_ref[...] = jnp.exp(ent)


ROWS_BT_PER_TILE = 512


def _vq_distance_argmin(z_bt, embed, counts_in, commit_in, total_flat_rows):
    nb, ld = z_bt.shape
    k, dc = embed.shape
    g = nb // ROWS_BT_PER_TILE
    idx, counts, commit, ppl = pl.pallas_call(
        functools.partial(_vq_tc_body, total_flat_rows),
        grid=(g,),
        in_specs=[
            pl.BlockSpec((ROWS_BT_PER_TILE, ld), lambda i: (i, 0)),
            pl.BlockSpec((k, dc), lambda i: (0, 0)),
            pl.BlockSpec((1, k), lambda i: (0, 0)),
            pl.BlockSpec((1, 1), lambda i: (0, 0)),
        ],
        out_specs=[
            pl.BlockSpec((ROWS_BT_PER_TILE, NUM_CODEBOOKS), lambda i: (i, 0)),
            pl.BlockSpec((1, k), lambda i: (0, 0)),
            pl.BlockSpec((1, 1), lambda i: (0, 0)),
            pl.BlockSpec((1, 1), lambda i: (0, 0)),
        ],
        out_shape=[
            jax.ShapeDtypeStruct((nb, NUM_CODEBOOKS), jnp.int32),
            jax.ShapeDtypeStruct((1, k), jnp.float32),
            jax.ShapeDtypeStruct((1, 1), jnp.float32),
            jax.ShapeDtypeStruct((1, 1), jnp.float32),
        ],
        scratch_shapes=[
            pltpu.VMEM((1, k), jnp.float32),
            pltpu.SMEM((1,), jnp.float32),
        ],
        compiler_params=pltpu.CompilerParams(
            dimension_semantics=("arbitrary",)),
    )(z_bt, embed, counts_in, commit_in)
    return idx, counts, commit, ppl


_SC_ROWS = 32


def _make_sc_gather(nb, ld, dc):
    info = plsc.get_sparse_core_info()
    nw = info.num_cores * info.num_subcores
    rows_per_w = nb // nw
    n_ch = rows_per_w // _SC_ROWS
    mrows = (nb // 128) // nw
    ncb = ld // dc
    mesh = plsc.VectorSubcoreMesh(core_axis_name="c", subcore_axis_name="s")

    @functools.partial(
        pl.kernel, mesh=mesh,
        out_type=jax.ShapeDtypeStruct((nb, ld), jnp.float32),
        scratch_types=[
            pltpu.VMEM((ncb, mrows, 128), jnp.int32),
            pltpu.VMEM((_SC_ROWS, ld), jnp.float32),
            pltpu.VMEM((_SC_ROWS, ld), jnp.float32),
            pltpu.SemaphoreType.DMA,
            pltpu.SemaphoreType.DMA,
        ],
    )
    def _gather(idx_hbm, table_hbm, out_hbm, idx_v, buf_a, buf_b, sem_a,
                sem_b):
        wid = lax.axis_index("s") * info.num_cores + lax.axis_index("c")
        base = wid * rows_per_w
        pltpu.sync_copy(idx_hbm.at[:, pl.ds(mrows * wid, mrows), :], idx_v)

        def chunk_gathers(m, buf, sem):
            j = m // 4
            off = (m % 4) * _SC_ROWS
            return [
                pltpu.async_copy(
                    table_hbm.at[idx_v.at[c, j, pl.ds(off, _SC_ROWS)]],
                    buf.at[:, pl.ds(c * dc, dc)], sem)
                for c in range(ncb)
            ]

        def body(t, carry):
            m0 = 2 * t
            cps_a = chunk_gathers(m0, buf_a, sem_a)
            cps_b = chunk_gathers(m0 + 1, buf_b, sem_b)
            for cp in cps_a:
                cp.wait()
            pltpu.sync_copy(buf_a,
                            out_hbm.at[pl.ds(base + m0 * _SC_ROWS, _SC_ROWS)])
            for cp in cps_b:
                cp.wait()
            pltpu.sync_copy(
                buf_b, out_hbm.at[pl.ds(base + (m0 + 1) * _SC_ROWS,
                                        _SC_ROWS)])
            return carry

        lax.fori_loop(0, n_ch // 2, body, 0)

    return _gather


def kernel(z_bt, embed):
    k, dc = embed.shape
    nb, ld = z_bt.shape
    half = nb // 2
    total_flat_rows = nb * (ld // dc)

    zero_counts = jnp.zeros((1, k), jnp.float32)
    zero_commit = jnp.zeros((1, 1), jnp.float32)
    idx1, c1, s1, _ = _vq_distance_argmin(
        z_bt[:half], embed, zero_counts, zero_commit, total_flat_rows)
    idx1_t = jnp.transpose(idx1).reshape(ld // dc, half // 128, 128)
    sc_gather = _make_sc_gather(half, ld, dc)
    q1 = sc_gather(idx1_t, embed)

    idx2, _, commit, ppl = _vq_distance_argmin(
        z_bt[half:], embed, c1, s1, total_flat_rows)
    idx2_t = jnp.transpose(idx2).reshape(ld // dc, half // 128, 128)
    q2 = sc_gather(idx2_t, embed)

    quantized_st = jnp.concatenate([q1, q2], axis=0)
    indices = jnp.concatenate([idx1, idx2], axis=0)

    commitment_loss = commit[0, 0] / z_bt.size
    codebook_loss = jnp.zeros((), dtype=z_bt.dtype)
    loss = COMMITMENT_COST * commitment_loss
    return (quantized_st, indices, loss, codebook_loss, commitment_loss,
            ppl[0, 0])


# speedup vs baseline: 1.4272x; 1.4272x over previous
import functools

import jax
import jax.numpy as jnp
from jax import lax
from jax.experimental import pallas as pl
from jax.experimental.pallas import tpu as pltpu
from jax.experimental.pallas import tpu_sc as plsc

NUM_EMBEDDINGS = 1024
CODE_DIM = 256
NUM_CODEBOOKS = 4
COMMITMENT_COST = 0.25
EPS = 1e-10

def _vq_tc_body(z_ref, e_ref, idx_ref, idxt_ref, commit_ref, ppl_ref,
                counts_scr, commit_scr, e2_scr):
    i = pl.program_id(0)
    e = e_ref[...]
    K, Dc = e.shape
    Rb = z_ref.shape[0]

    @pl.when(i == 0)
    def _pre():
        e2_scr[...] = jnp.sum(e * e, axis=1, keepdims=True).reshape(1, -1)

    e2 = e2_scr[...]
    iota_f = lax.broadcasted_iota(jnp.int32, (1, K), 1).astype(jnp.float32)
    ones_r = jnp.ones((1, Rb), jnp.float32)
    idx_cols = []
    tile_counts = jnp.zeros((1, K), jnp.float32)
    tile_commit = jnp.zeros((), jnp.float32)
    for c in range(NUM_CODEBOOKS):
        zc = z_ref[:, c * Dc:(c + 1) * Dc]
        z2 = jnp.sum(zc * zc, axis=1, keepdims=True)
        s_neg = lax.dot_general(zc * -2.0, e, (((1,), (1,)), ((), ())),
                                preferred_element_type=jnp.float32)
        d = (z2 + s_neg) + e2
        md = jnp.min(d, axis=1, keepdims=True)
        idxf = jnp.min(jnp.where(d == md, iota_f, 2048.0), axis=1,
                       keepdims=True)
        idxi = idxf.astype(jnp.int32)
        idx_cols.append(idxi)
        idxt_ref[:, c, :] = idxi.reshape(Rb // 128, 128)
        onehot = (iota_f == idxf).astype(jnp.float32)
        tile_counts += lax.dot_general(ones_r, onehot,
                                       (((1,), (0,)), ((), ())),
                                       preferred_element_type=jnp.float32)
        tile_commit += jnp.sum(md)
    idx_ref[...] = jnp.concatenate(idx_cols, axis=1)

    @pl.when(i == 0)
    def _init():
        counts_scr[...] = tile_counts
        commit_scr[0] = tile_commit

    @pl.when(i > 0)
    def _acc():
        counts_scr[...] += tile_counts
        commit_scr[0] += tile_commit

    @pl.when(i == pl.num_programs(0) - 1)
    def _fin():
        total_rows = Rb * NUM_CODEBOOKS * pl.num_programs(0)
        p = counts_scr[...] / total_rows
        ent = -jnp.sum(p * jnp.log(p + EPS), axis=1, keepdims=True)
        ppl_ref[...] = jnp.exp(ent)
        commit_ref[...] = jnp.full((1, 1), commit_scr[0], jnp.float32)


ROWS_BT_PER_TILE = 512


def _vq_distance_argmin(z_bt, embed):
    nb, ld = z_bt.shape
    k, dc = embed.shape
    g = nb // ROWS_BT_PER_TILE
    rt = ROWS_BT_PER_TILE // 128
    idx, idxt, commit, ppl = pl.pallas_call(
        _vq_tc_body,
        grid=(g,),
        in_specs=[
            pl.BlockSpec((ROWS_BT_PER_TILE, ld), lambda i: (i, 0)),
            pl.BlockSpec((k, dc), lambda i: (0, 0)),
        ],
        out_specs=[
            pl.BlockSpec((ROWS_BT_PER_TILE, NUM_CODEBOOKS), lambda i: (i, 0)),
            pl.BlockSpec((rt, NUM_CODEBOOKS, 128), lambda i: (i, 0, 0)),
            pl.BlockSpec((1, 1), lambda i: (0, 0)),
            pl.BlockSpec((1, 1), lambda i: (0, 0)),
        ],
        out_shape=[
            jax.ShapeDtypeStruct((nb, NUM_CODEBOOKS), jnp.int32),
            jax.ShapeDtypeStruct((nb // 128, NUM_CODEBOOKS, 128), jnp.int32),
            jax.ShapeDtypeStruct((1, 1), jnp.float32),
            jax.ShapeDtypeStruct((1, 1), jnp.float32),
        ],
        scratch_shapes=[
            pltpu.VMEM((1, k), jnp.float32),
            pltpu.SMEM((1,), jnp.float32),
            pltpu.VMEM((1, k), jnp.float32),
        ],
        compiler_params=pltpu.CompilerParams(
            dimension_semantics=("arbitrary",)),
    )(z_bt, embed)
    return idx, idxt, commit[0, 0], ppl[0, 0]


_SC_ROWS = 32


def _make_sc_gather(nb, ld, dc):
    info = plsc.get_sparse_core_info()
    nw = info.num_cores * info.num_subcores
    rows_per_w = nb // nw
    n_ch = rows_per_w // _SC_ROWS
    ncb = ld // dc
    mesh = plsc.VectorSubcoreMesh(core_axis_name="c", subcore_axis_name="s")

    @functools.partial(
        pl.kernel, mesh=mesh,
        out_type=jax.ShapeDtypeStruct((nb, ld), jnp.float32),
        scratch_types=[
            pltpu.VMEM((nb // 128 // nw, ncb, 128), jnp.int32),
            pltpu.VMEM((_SC_ROWS, ld), jnp.float32),
            pltpu.VMEM((_SC_ROWS, ld), jnp.float32),
            pltpu.SemaphoreType.DMA,
            pltpu.SemaphoreType.DMA,
        ],
    )
    def _gather(idx_hbm, table_hbm, out_hbm, idx_v, buf_a, buf_b, sem_a,
                sem_b):
        wid = lax.axis_index("s") * info.num_cores + lax.axis_index("c")
        base = wid * rows_per_w
        mrows = nb // 128 // nw
        pltpu.sync_copy(idx_hbm.at[pl.ds(mrows * wid, mrows)], idx_v)

        def chunk_gathers(m, buf, sem):
            j = m // 4
            off = (m % 4) * _SC_ROWS
            return [
                pltpu.async_copy(
                    table_hbm.at[idx_v.at[j, c, pl.ds(off, _SC_ROWS)]],
                    buf.at[:, pl.ds(c * dc, dc)], sem)
                for c in range(ncb)
            ]

        def body(t, carry):
            m0 = 2 * t
            cps_a = chunk_gathers(m0, buf_a, sem_a)
            cps_b = chunk_gathers(m0 + 1, buf_b, sem_b)
            for cp in cps_a:
                cp.wait()
            pltpu.sync_copy(buf_a,
                            out_hbm.at[pl.ds(base + m0 * _SC_ROWS, _SC_ROWS)])
            for cp in cps_b:
                cp.wait()
            pltpu.sync_copy(
                buf_b, out_hbm.at[pl.ds(base + (m0 + 1) * _SC_ROWS,
                                        _SC_ROWS)])
            return carry

        lax.fori_loop(0, n_ch // 2, body, 0)

    return _gather


def kernel(z_bt, embed):
    k, dc = embed.shape
    nb, ld = z_bt.shape

    indices, idx_t, commit_sum, perplexity = _vq_distance_argmin(z_bt, embed)

    quantized_st = _make_sc_gather(nb, ld, dc)(idx_t, embed)

    commitment_loss = commit_sum / z_bt.size
    codebook_loss = jnp.zeros((), dtype=z_bt.dtype)
    loss = COMMITMENT_COST * commitment_loss
    return (quantized_st, indices, loss, codebook_loss, commitment_loss,
            perplexity)


# speedup vs baseline: 1.5306x; 1.0725x over previous
import functools

import jax
import jax.numpy as jnp
from jax import lax
from jax.experimental import pallas as pl
from jax.experimental.pallas import tpu as pltpu
from jax.experimental.pallas import tpu_sc as plsc

NUM_EMBEDDINGS = 1024
CODE_DIM = 256
NUM_CODEBOOKS = 4
COMMITMENT_COST = 0.25
EPS = 1e-10

def _vq_tc_body(z_ref, e_ref, idx_ref, idxt_ref, commit_ref, ppl_ref,
                counts_scr, commit_scr, e2_scr):
    i = pl.program_id(0)
    e = e_ref[...]
    K, Dc = e.shape
    Rb = z_ref.shape[0]

    @pl.when(i == 0)
    def _pre():
        e2_scr[...] = jnp.sum(e * e, axis=1, keepdims=True).reshape(1, -1)

    e2 = e2_scr[...]
    iota_f = lax.broadcasted_iota(jnp.int32, (1, K), 1).astype(jnp.float32)
    ones_r = jnp.ones((1, Rb), jnp.float32)
    idx_cols = []
    tile_counts = jnp.zeros((1, K), jnp.float32)
    tile_commit = jnp.zeros((), jnp.float32)
    for c in range(NUM_CODEBOOKS):
        zc = z_ref[:, c * Dc:(c + 1) * Dc]
        z2 = jnp.sum(zc * zc, axis=1, keepdims=True)
        s_neg = lax.dot_general(zc * -2.0, e, (((1,), (1,)), ((), ())),
                                preferred_element_type=jnp.float32)
        d = (z2 + s_neg) + e2
        md = jnp.min(d, axis=1, keepdims=True)
        idxf = jnp.min(jnp.where(d == md, iota_f, 2048.0), axis=1,
                       keepdims=True)
        idxi = idxf.astype(jnp.int32)
        idx_cols.append(idxi)
        idxt_ref[:, c, :] = idxi.reshape(Rb // 128, 128)
        onehot = (iota_f == idxf).astype(jnp.float32)
        tile_counts += lax.dot_general(ones_r, onehot,
                                       (((1,), (0,)), ((), ())),
                                       preferred_element_type=jnp.float32)
        tile_commit += jnp.sum(md)
    idx_ref[...] = jnp.concatenate(idx_cols, axis=1)

    @pl.when(i == 0)
    def _init():
        counts_scr[...] = tile_counts
        commit_scr[0] = tile_commit

    @pl.when(i > 0)
    def _acc():
        counts_scr[...] += tile_counts
        commit_scr[0] += tile_commit

    @pl.when(i == pl.num_programs(0) - 1)
    def _fin():
        total_rows = Rb * NUM_CODEBOOKS * pl.num_programs(0)
        p = counts_scr[...] / total_rows
        ent = -jnp.sum(p * jnp.log(p + EPS), axis=1, keepdims=True)
        ppl_ref[...] = jnp.exp(ent)
        commit_ref[...] = jnp.full((1, 1), commit_scr[0], jnp.float32)


ROWS_BT_PER_TILE = 1024


def _vq_distance_argmin(z_bt, embed):
    nb, ld = z_bt.shape
    k, dc = embed.shape
    g = nb // ROWS_BT_PER_TILE
    rt = ROWS_BT_PER_TILE // 128
    idx, idxt, commit, ppl = pl.pallas_call(
        _vq_tc_body,
        grid=(g,),
        in_specs=[
            pl.BlockSpec((ROWS_BT_PER_TILE, ld), lambda i: (i, 0)),
            pl.BlockSpec((k, dc), lambda i: (0, 0)),
        ],
        out_specs=[
            pl.BlockSpec((ROWS_BT_PER_TILE, NUM_CODEBOOKS), lambda i: (i, 0)),
            pl.BlockSpec((rt, NUM_CODEBOOKS, 128), lambda i: (i, 0, 0)),
            pl.BlockSpec((1, 1), lambda i: (0, 0)),
            pl.BlockSpec((1, 1), lambda i: (0, 0)),
        ],
        out_shape=[
            jax.ShapeDtypeStruct((nb, NUM_CODEBOOKS), jnp.int32),
            jax.ShapeDtypeStruct((nb // 128, NUM_CODEBOOKS, 128), jnp.int32),
            jax.ShapeDtypeStruct((1, 1), jnp.float32),
            jax.ShapeDtypeStruct((1, 1), jnp.float32),
        ],
        scratch_shapes=[
            pltpu.VMEM((1, k), jnp.float32),
            pltpu.SMEM((1,), jnp.float32),
            pltpu.VMEM((1, k), jnp.float32),
        ],
        compiler_params=pltpu.CompilerParams(
            dimension_semantics=("arbitrary",)),
    )(z_bt, embed)
    return idx, idxt, commit[0, 0], ppl[0, 0]


_SC_ROWS = 32


def _make_sc_gather(nb, ld, dc):
    info = plsc.get_sparse_core_info()
    nw = info.num_cores * info.num_subcores
    rows_per_w = nb // nw
    n_ch = rows_per_w // _SC_ROWS
    ncb = ld // dc
    mesh = plsc.VectorSubcoreMesh(core_axis_name="c", subcore_axis_name="s")

    @functools.partial(
        pl.kernel, mesh=mesh,
        out_type=jax.ShapeDtypeStruct((nb, ld), jnp.float32),
        scratch_types=[
            pltpu.VMEM((nb // 128 // nw, ncb, 128), jnp.int32),
            pltpu.VMEM((_SC_ROWS, ld), jnp.float32),
            pltpu.VMEM((_SC_ROWS, ld), jnp.float32),
            pltpu.SemaphoreType.DMA,
            pltpu.SemaphoreType.DMA,
        ],
    )
    def _gather(idx_hbm, table_hbm, out_hbm, idx_v, buf_a, buf_b, sem_a,
                sem_b):
        wid = lax.axis_index("s") * info.num_cores + lax.axis_index("c")
        base = wid * rows_per_w
        mrows = nb // 128 // nw
        pltpu.sync_copy(idx_hbm.at[pl.ds(mrows * wid, mrows)], idx_v)

        def chunk_gathers(m, buf, sem):
            j = m // 4
            off = (m % 4) * _SC_ROWS
            return [
                pltpu.async_copy(
                    table_hbm.at[idx_v.at[j, c, pl.ds(off, _SC_ROWS)]],
                    buf.at[:, pl.ds(c * dc, dc)], sem)
                for c in range(ncb)
            ]

        def body(t, carry):
            m0 = 2 * t
            cps_a = chunk_gathers(m0, buf_a, sem_a)
            cps_b = chunk_gathers(m0 + 1, buf_b, sem_b)
            for cp in cps_a:
                cp.wait()
            pltpu.sync_copy(buf_a,
                            out_hbm.at[pl.ds(base + m0 * _SC_ROWS, _SC_ROWS)])
            for cp in cps_b:
                cp.wait()
            pltpu.sync_copy(
                buf_b, out_hbm.at[pl.ds(base + (m0 + 1) * _SC_ROWS,
                                        _SC_ROWS)])
            return carry

        lax.fori_loop(0, n_ch // 2, body, 0)

    return _gather


def kernel(z_bt, embed):
    k, dc = embed.shape
    nb, ld = z_bt.shape

    indices, idx_t, commit_sum, perplexity = _vq_distance_argmin(z_bt, embed)

    quantized_st = _make_sc_gather(nb, ld, dc)(idx_t, embed)

    commitment_loss = commit_sum / z_bt.size
    codebook_loss = jnp.zeros((), dtype=z_bt.dtype)
    loss = COMMITMENT_COST * commitment_loss
    return (quantized_st, indices, loss, codebook_loss, commitment_loss,
            perplexity)


# speedup vs baseline: 1.5697x; 1.0255x over previous
import functools

import jax
import jax.numpy as jnp
from jax import lax
from jax.experimental import pallas as pl
from jax.experimental.pallas import tpu as pltpu
from jax.experimental.pallas import tpu_sc as plsc

NUM_EMBEDDINGS = 1024
CODE_DIM = 256
NUM_CODEBOOKS = 4
COMMITMENT_COST = 0.25
EPS = 1e-10

def _vq_tc_body(z_ref, e_ref, idx_ref, idxt_ref, commit_ref, ppl_ref,
                counts_scr, commit_scr, e2_scr):
    i = pl.program_id(0)
    e = e_ref[...]
    K, Dc = e.shape
    Rb = z_ref.shape[0]

    @pl.when(i == 0)
    def _pre():
        e2_scr[...] = jnp.sum(e * e, axis=1, keepdims=True).reshape(1, -1)

    e2 = e2_scr[...]
    iota_f = lax.broadcasted_iota(jnp.int32, (1, K), 1).astype(jnp.float32)
    ones_r = jnp.ones((1, Rb), jnp.float32)
    idx_cols = []
    tile_counts = jnp.zeros((1, K), jnp.float32)
    tile_commit = jnp.zeros((), jnp.float32)
    for c in range(NUM_CODEBOOKS):
        zc = z_ref[:, c * Dc:(c + 1) * Dc]
        z2 = jnp.sum(zc * zc, axis=1, keepdims=True)
        s_neg = lax.dot_general(zc * -2.0, e, (((1,), (1,)), ((), ())),
                                preferred_element_type=jnp.float32)
        d = (z2 + s_neg) + e2
        md = jnp.min(d, axis=1, keepdims=True)
        idxf = jnp.min(jnp.where(d == md, iota_f, 2048.0), axis=1,
                       keepdims=True)
        idxi = idxf.astype(jnp.int32)
        idx_cols.append(idxi)
        idxt_ref[:, c, :] = idxi.reshape(Rb // 128, 128)
        onehot = (iota_f == idxf).astype(jnp.float32)
        tile_counts += lax.dot_general(ones_r, onehot,
                                       (((1,), (0,)), ((), ())),
                                       preferred_element_type=jnp.float32)
        tile_commit += jnp.sum(md)
    idx_ref[...] = jnp.concatenate(idx_cols, axis=1)

    @pl.when(i == 0)
    def _init():
        counts_scr[...] = tile_counts
        commit_scr[0] = tile_commit

    @pl.when(i > 0)
    def _acc():
        counts_scr[...] += tile_counts
        commit_scr[0] += tile_commit

    @pl.when(i == pl.num_programs(0) - 1)
    def _fin():
        total_rows = Rb * NUM_CODEBOOKS * pl.num_programs(0)
        p = counts_scr[...] / total_rows
        ent = -jnp.sum(p * jnp.log(p + EPS), axis=1, keepdims=True)
        ppl_ref[...] = jnp.exp(ent)
        commit_ref[...] = jnp.full((1, 1), commit_scr[0], jnp.float32)


ROWS_BT_PER_TILE = 2048


def _vq_distance_argmin(z_bt, embed):
    nb, ld = z_bt.shape
    k, dc = embed.shape
    g = nb // ROWS_BT_PER_TILE
    rt = ROWS_BT_PER_TILE // 128
    idx, idxt, commit, ppl = pl.pallas_call(
        _vq_tc_body,
        grid=(g,),
        in_specs=[
            pl.BlockSpec((ROWS_BT_PER_TILE, ld), lambda i: (i, 0)),
            pl.BlockSpec((k, dc), lambda i: (0, 0)),
        ],
        out_specs=[
            pl.BlockSpec((ROWS_BT_PER_TILE, NUM_CODEBOOKS), lambda i: (i, 0)),
            pl.BlockSpec((rt, NUM_CODEBOOKS, 128), lambda i: (i, 0, 0)),
            pl.BlockSpec((1, 1), lambda i: (0, 0)),
            pl.BlockSpec((1, 1), lambda i: (0, 0)),
        ],
        out_shape=[
            jax.ShapeDtypeStruct((nb, NUM_CODEBOOKS), jnp.int32),
            jax.ShapeDtypeStruct((nb // 128, NUM_CODEBOOKS, 128), jnp.int32),
            jax.ShapeDtypeStruct((1, 1), jnp.float32),
            jax.ShapeDtypeStruct((1, 1), jnp.float32),
        ],
        scratch_shapes=[
            pltpu.VMEM((1, k), jnp.float32),
            pltpu.SMEM((1,), jnp.float32),
            pltpu.VMEM((1, k), jnp.float32),
        ],
        compiler_params=pltpu.CompilerParams(
            dimension_semantics=("arbitrary",)),
    )(z_bt, embed)
    return idx, idxt, commit[0, 0], ppl[0, 0]


_SC_ROWS = 32


def _make_sc_gather(nb, ld, dc):
    info = plsc.get_sparse_core_info()
    nw = info.num_cores * info.num_subcores
    rows_per_w = nb // nw
    n_ch = rows_per_w // _SC_ROWS
    ncb = ld // dc
    mesh = plsc.VectorSubcoreMesh(core_axis_name="c", subcore_axis_name="s")

    @functools.partial(
        pl.kernel, mesh=mesh,
        out_type=jax.ShapeDtypeStruct((nb, ld), jnp.float32),
        scratch_types=[
            pltpu.VMEM((nb // 128 // nw, ncb, 128), jnp.int32),
            pltpu.VMEM((_SC_ROWS, ld), jnp.float32),
            pltpu.VMEM((_SC_ROWS, ld), jnp.float32),
            pltpu.SemaphoreType.DMA,
            pltpu.SemaphoreType.DMA,
        ],
    )
    def _gather(idx_hbm, table_hbm, out_hbm, idx_v, buf_a, buf_b, sem_a,
                sem_b):
        wid = lax.axis_index("s") * info.num_cores + lax.axis_index("c")
        base = wid * rows_per_w
        mrows = nb // 128 // nw
        pltpu.sync_copy(idx_hbm.at[pl.ds(mrows * wid, mrows)], idx_v)

        def chunk_gathers(m, buf, sem):
            j = m // 4
            off = (m % 4) * _SC_ROWS
            return [
                pltpu.async_copy(
                    table_hbm.at[idx_v.at[j, c, pl.ds(off, _SC_ROWS)]],
                    buf.at[:, pl.ds(c * dc, dc)], sem)
                for c in range(ncb)
            ]

        def body(t, carry):
            m0 = 2 * t
            cps_a = chunk_gathers(m0, buf_a, sem_a)
            cps_b = chunk_gathers(m0 + 1, buf_b, sem_b)
            for cp in cps_a:
                cp.wait()
            pltpu.sync_copy(buf_a,
                            out_hbm.at[pl.ds(base + m0 * _SC_ROWS, _SC_ROWS)])
            for cp in cps_b:
                cp.wait()
            pltpu.sync_copy(
                buf_b, out_hbm.at[pl.ds(base + (m0 + 1) * _SC_ROWS,
                                        _SC_ROWS)])
            return carry

        lax.fori_loop(0, n_ch // 2, body, 0)

    return _gather


def kernel(z_bt, embed):
    k, dc = embed.shape
    nb, ld = z_bt.shape

    indices, idx_t, commit_sum, perplexity = _vq_distance_argmin(z_bt, embed)

    quantized_st = _make_sc_gather(nb, ld, dc)(idx_t, embed)

    commitment_loss = commit_sum / z_bt.size
    codebook_loss = jnp.zeros((), dtype=z_bt.dtype)
    loss = COMMITMENT_COST * commitment_loss
    return (quantized_st, indices, loss, codebook_loss, commitment_loss,
            perplexity)


# speedup vs baseline: 1.5768x; 1.0046x over previous
import functools

import jax
import jax.numpy as jnp
from jax import lax
from jax.experimental import pallas as pl
from jax.experimental.pallas import tpu as pltpu
from jax.experimental.pallas import tpu_sc as plsc

NUM_EMBEDDINGS = 1024
CODE_DIM = 256
NUM_CODEBOOKS = 4
COMMITMENT_COST = 0.25
EPS = 1e-10

def _vq_tc_body(z_ref, e_ref, idx_ref, idxt_ref, commit_ref, ppl_ref,
                counts_scr, commit_scr, e2_scr):
    i = pl.program_id(0)
    e = e_ref[...]
    K, Dc = e.shape
    Rb = z_ref.shape[0]

    @pl.when(i == 0)
    def _pre():
        e2_scr[...] = jnp.sum(e * e, axis=1, keepdims=True).reshape(1, -1)

    e2 = e2_scr[...]
    iota_f = lax.broadcasted_iota(jnp.int32, (1, K), 1).astype(jnp.float32)
    ones_r = jnp.ones((1, Rb), jnp.bfloat16)
    idx_cols = []
    tile_counts = jnp.zeros((1, K), jnp.float32)
    tile_commit = jnp.zeros((), jnp.float32)
    for c in range(NUM_CODEBOOKS):
        zc = z_ref[:, c * Dc:(c + 1) * Dc]
        z2 = jnp.sum(zc * zc, axis=1, keepdims=True)
        s_neg = lax.dot_general(zc * -2.0, e, (((1,), (1,)), ((), ())),
                                preferred_element_type=jnp.float32)
        d = (z2 + s_neg) + e2
        md = jnp.min(d, axis=1, keepdims=True)
        idxf = jnp.min(jnp.where(d == md, iota_f, 2048.0), axis=1,
                       keepdims=True)
        idxi = idxf.astype(jnp.int32)
        idx_cols.append(idxi)
        idxt_ref[:, c, :] = idxi.reshape(Rb // 128, 128)
        onehot = (iota_f == idxf).astype(jnp.bfloat16)
        tile_counts += lax.dot_general(ones_r, onehot,
                                       (((1,), (0,)), ((), ())),
                                       preferred_element_type=jnp.float32)
        tile_commit += jnp.sum(md)
    idx_ref[...] = jnp.concatenate(idx_cols, axis=1)

    @pl.when(i == 0)
    def _init():
        counts_scr[...] = tile_counts
        commit_scr[0] = tile_commit

    @pl.when(i > 0)
    def _acc():
        counts_scr[...] += tile_counts
        commit_scr[0] += tile_commit

    @pl.when(i == pl.num_programs(0) - 1)
    def _fin():
        total_rows = Rb * NUM_CODEBOOKS * pl.num_programs(0)
        p = counts_scr[...] / total_rows
        ent = -jnp.sum(p * jnp.log(p + EPS), axis=1, keepdims=True)
        ppl_ref[...] = jnp.exp(ent)
        commit_ref[...] = jnp.full((1, 1), commit_scr[0], jnp.float32)


ROWS_BT_PER_TILE = 2048


def _vq_distance_argmin(z_bt, embed):
    nb, ld = z_bt.shape
    k, dc = embed.shape
    g = nb // ROWS_BT_PER_TILE
    rt = ROWS_BT_PER_TILE // 128
    idx, idxt, commit, ppl = pl.pallas_call(
        _vq_tc_body,
        grid=(g,),
        in_specs=[
            pl.BlockSpec((ROWS_BT_PER_TILE, ld), lambda i: (i, 0)),
            pl.BlockSpec((k, dc), lambda i: (0, 0)),
        ],
        out_specs=[
            pl.BlockSpec((ROWS_BT_PER_TILE, NUM_CODEBOOKS), lambda i: (i, 0)),
            pl.BlockSpec((rt, NUM_CODEBOOKS, 128), lambda i: (i, 0, 0)),
            pl.BlockSpec((1, 1), lambda i: (0, 0)),
            pl.BlockSpec((1, 1), lambda i: (0, 0)),
        ],
        out_shape=[
            jax.ShapeDtypeStruct((nb, NUM_CODEBOOKS), jnp.int32),
            jax.ShapeDtypeStruct((nb // 128, NUM_CODEBOOKS, 128), jnp.int32),
            jax.ShapeDtypeStruct((1, 1), jnp.float32),
            jax.ShapeDtypeStruct((1, 1), jnp.float32),
        ],
        scratch_shapes=[
            pltpu.VMEM((1, k), jnp.float32),
            pltpu.SMEM((1,), jnp.float32),
            pltpu.VMEM((1, k), jnp.float32),
        ],
        compiler_params=pltpu.CompilerParams(
            dimension_semantics=("arbitrary",)),
    )(z_bt, embed)
    return idx, idxt, commit[0, 0], ppl[0, 0]


_SC_ROWS = 32


def _make_sc_gather(nb, ld, dc):
    info = plsc.get_sparse_core_info()
    nw = info.num_cores * info.num_subcores
    rows_per_w = nb // nw
    n_ch = rows_per_w // _SC_ROWS
    ncb = ld // dc
    mesh = plsc.VectorSubcoreMesh(core_axis_name="c", subcore_axis_name="s")

    @functools.partial(
        pl.kernel, mesh=mesh,
        out_type=jax.ShapeDtypeStruct((nb, ld), jnp.float32),
        scratch_types=[
            pltpu.VMEM((nb // 128 // nw, ncb, 128), jnp.int32),
            pltpu.VMEM((_SC_ROWS, ld), jnp.float32),
            pltpu.VMEM((_SC_ROWS, ld), jnp.float32),
            pltpu.SemaphoreType.DMA,
            pltpu.SemaphoreType.DMA,
            pltpu.SemaphoreType.DMA,
            pltpu.SemaphoreType.DMA,
        ],
    )
    def _gather(idx_hbm, table_hbm, out_hbm, idx_v, buf_a, buf_b, sem_a,
                sem_b, wsem_a, wsem_b):
        wid = lax.axis_index("s") * info.num_cores + lax.axis_index("c")
        base = wid * rows_per_w
        mrows = nb // 128 // nw
        pltpu.sync_copy(idx_hbm.at[pl.ds(mrows * wid, mrows)], idx_v)

        cpr = 128 // _SC_ROWS

        def chunk_gathers(m, buf, sem):
            j = m // cpr
            off = (m % cpr) * _SC_ROWS
            return [
                pltpu.async_copy(
                    table_hbm.at[idx_v.at[j, c, pl.ds(off, _SC_ROWS)]],
                    buf.at[:, pl.ds(c * dc, dc)], sem)
                for c in range(ncb)
            ]

        bufs = (buf_a, buf_b)
        gsems = (sem_a, sem_b)
        wsems = (wsem_a, wsem_b)
        pending_g = [chunk_gathers(0, buf_a, sem_a),
                     chunk_gathers(1, buf_b, sem_b)]
        final_wbs = []
        for m in range(n_ch):
            b = m % 2
            for cp in pending_g[b]:
                cp.wait()
            wb = pltpu.async_copy(
                bufs[b], out_hbm.at[pl.ds(base + m * _SC_ROWS, _SC_ROWS)],
                wsems[b])
            if m + 2 < n_ch:
                wb.wait()
                pending_g[b] = chunk_gathers(m + 2, bufs[b], gsems[b])
            else:
                final_wbs.append(wb)
        for wb in final_wbs:
            wb.wait()

    return _gather


def kernel(z_bt, embed):
    k, dc = embed.shape
    nb, ld = z_bt.shape

    indices, idx_t, commit_sum, perplexity = _vq_distance_argmin(z_bt, embed)

    quantized_st = _make_sc_gather(nb, ld, dc)(idx_t, embed)

    commitment_loss = commit_sum / z_bt.size
    codebook_loss = jnp.zeros((), dtype=z_bt.dtype)
    loss = COMMITMENT_COST * commitment_loss
    return (quantized_st, indices, loss, codebook_loss, commitment_loss,
            perplexity)
